# Initial kernel scaffold; baseline (speedup 1.0000x reference)
import jax
import jax.numpy as jnp
from jax.experimental import pallas as pl


def kernel(x, codebook):
    raise NotImplementedError("write your pallas kernel here")


# speedup vs baseline: 2.7004x; 2.7004x over previous
import functools

import jax
import jax.numpy as jnp
from jax import lax
from jax.experimental import pallas as pl

K = 1024
E = 256
B = 8
T = 2048
TB = 512


def _lane_fold(v):
    acc = v[:, 0:128]
    for o in range(128, v.shape[1], 128):
        acc = acc + v[:, o:o + 128]
    return acc


def _vq_kernel(x_ref, cb_ref, out_ref, idx_ref, smin_ref, scom_ref,
               sx_ref, sx2_ref):
    b = pl.program_id(0)
    j = pl.program_id(1)

    @pl.when(jnp.logical_and(b == 0, j == 0))
    def _():
        smin_ref[...] = jnp.zeros_like(smin_ref)
        scom_ref[...] = jnp.zeros_like(scom_ref)
        sx_ref[...] = jnp.zeros_like(sx_ref)
        sx2_ref[...] = jnp.zeros_like(sx2_ref)

    x = x_ref[0]
    cb = cb_ref[...]

    m = lax.dot_general(cb, x, (((1,), (0,)), ((), ())),
                        preferred_element_type=jnp.float32)
    cb2 = jnp.sum(cb * cb, axis=1, keepdims=True)
    x2 = jnp.sum(x * x, axis=0, keepdims=True)
    d = (x2 - 2.0 * m) + cb2

    min_d = jnp.min(d, axis=0, keepdims=True)
    iota = lax.broadcasted_iota(jnp.int32, d.shape, 0)
    idx = jnp.min(jnp.where(d == min_d, iota, K), axis=0, keepdims=True)

    onehot = (iota == idx).astype(jnp.float32)
    g = lax.dot_general(cb, onehot, (((0,), (0,)), ((), ())),
                        preferred_element_type=jnp.float32)

    out_ref[0] = x + (g - x)
    idx_ref[0] = idx

    smin_ref[...] += _lane_fold(min_d)
    scom_ref[...] += _lane_fold(jnp.sum((g - x) * (g - x), axis=0,
                                        keepdims=True))
    sx_ref[...] += _lane_fold(jnp.sum(x, axis=0, keepdims=True))
    sx2_ref[...] += _lane_fold(x2)


@jax.jit
def kernel(x, codebook):
    n_elem = B * E * T
    grid = (B, T // TB)
    out, idx, smin, scom, sx, sx2 = pl.pallas_call(
        _vq_kernel,
        grid=grid,
        in_specs=[
            pl.BlockSpec((1, E, TB), lambda b, j: (b, 0, j)),
            pl.BlockSpec((K, E), lambda b, j: (0, 0)),
        ],
        out_specs=[
            pl.BlockSpec((1, E, TB), lambda b, j: (b, 0, j)),
            pl.BlockSpec((1, 1, TB), lambda b, j: (b, 0, j)),
            pl.BlockSpec((1, 128), lambda b, j: (0, 0)),
            pl.BlockSpec((1, 128), lambda b, j: (0, 0)),
            pl.BlockSpec((1, 128), lambda b, j: (0, 0)),
            pl.BlockSpec((1, 128), lambda b, j: (0, 0)),
        ],
        out_shape=[
            jax.ShapeDtypeStruct((B, E, T), jnp.float32),
            jax.ShapeDtypeStruct((B, 1, T), jnp.int32),
            jax.ShapeDtypeStruct((1, 128), jnp.float32),
            jax.ShapeDtypeStruct((1, 128), jnp.float32),
            jax.ShapeDtypeStruct((1, 128), jnp.float32),
            jax.ShapeDtypeStruct((1, 128), jnp.float32),
        ],
    )(x, codebook)

    fit = jnp.sum(smin) / (B * T)
    commit_loss = jnp.sum(scom) / n_elem
    mean = jnp.sum(sx) / n_elem
    x_norm = jnp.sqrt(jnp.maximum(jnp.sum(sx2) / n_elem - mean * mean, 0.0))
    codebook_idxs = idx.reshape(B, T)
    return (out, commit_loss, fit, x_norm, codebook_idxs)
